# Initial kernel scaffold; baseline (speedup 1.0000x reference)
import jax
import jax.numpy as jnp
from jax.experimental import pallas as pl


def kernel(x, day_embed, hour_embed, minute_embed, second_embed):
    raise NotImplementedError("write your pallas kernel here")


# speedup vs baseline: 6.7641x; 6.7641x over previous
import functools

import jax
import jax.numpy as jnp
from jax import lax
from jax.experimental import pallas as pl
from jax.experimental.pallas import tpu as pltpu

B, L, D = 4096, 200, 64
BL = B * L

OFF_DAY, OFF_HOUR, OFF_MIN, OFF_SEC = 0, 32, 56, 116
KDIM = 256

BLK = 4096


def _body(x_ref, w_ref, o_ref):
    idx = x_ref[...]
    iota = lax.broadcasted_iota(jnp.int32, (BLK, KDIM), 1)
    hit = (
        (iota == idx[:, 0:1] + OFF_DAY)
        | (iota == idx[:, 1:2] + OFF_HOUR)
        | (iota == idx[:, 2:3] + OFF_MIN)
        | (iota == idx[:, 3:4] + OFF_SEC)
    )
    mh = hit.astype(jnp.float32)
    o_ref[...] = jnp.dot(mh, w_ref[...], preferred_element_type=jnp.float32)


@jax.jit
def kernel(x, day_embed, hour_embed, minute_embed, second_embed):
    xf = x.astype(jnp.int32).reshape(BL, 4)
    w = jnp.zeros((KDIM, D), jnp.float32)
    w = w.at[OFF_DAY:OFF_DAY + 32].set(day_embed)
    w = w.at[OFF_HOUR:OFF_HOUR + 24].set(hour_embed)
    w = w.at[OFF_MIN:OFF_MIN + 60].set(minute_embed)
    w = w.at[OFF_SEC:OFF_SEC + 60].set(second_embed)

    out = pl.pallas_call(
        _body,
        grid=(BL // BLK,),
        in_specs=[
            pl.BlockSpec((BLK, 4), lambda i: (i, 0)),
            pl.BlockSpec((KDIM, D), lambda i: (0, 0)),
        ],
        out_specs=pl.BlockSpec((BLK, D), lambda i: (i, 0)),
        out_shape=jax.ShapeDtypeStruct((BL, D), jnp.float32),
    )(xf, w)
    return out.reshape(B, L, D)
